# baseline (speedup 1.0000x reference)
import functools

import jax
import jax.numpy as jnp
from jax import lax
from jax.experimental import pallas as pl
from jax.experimental.pallas import tpu as pltpu
from jax.experimental.pallas import tpu_sc as plsc

NC = 2
NS = 16
NW = NC * NS

ROWS = 4096 * 200
D = 64
CHUNK = 128
ROWS_PER_W = ROWS // NW
NCHUNK = ROWS_PER_W // CHUNK


def _make_kernel():
  mesh = plsc.VectorSubcoreMesh(core_axis_name="c", subcore_axis_name="s")

  @functools.partial(
      pl.kernel,
      out_type=jax.ShapeDtypeStruct((ROWS, D), jnp.float32),
      mesh=mesh,
      compiler_params=pltpu.CompilerParams(use_tc_tiling_on_sc=False),
      scratch_types=[
          pltpu.VMEM((NCHUNK, CHUNK), jnp.int32),
          pltpu.VMEM((CHUNK, D), jnp.float32),
          pltpu.VMEM((CHUNK, D), jnp.float32),
          pltpu.SemaphoreType.DMA,
          pltpu.SemaphoreType.DMA,
      ],
  )
  def k(idx_hbm, table_hbm, out_hbm, idx_v, rows0, rows1, sem0, sem1):
    wid = lax.axis_index("s") * NC + lax.axis_index("c")
    base = wid * ROWS_PER_W
    pltpu.sync_copy(idx_hbm.at[pl.ds(wid * NCHUNK, NCHUNK)], idx_v)

    bufs = (rows0, rows1)
    sems = (sem0, sem1)

    pltpu.async_copy(table_hbm.at[idx_v.at[0]], rows0, sem0)

    def step(j, _):
      @pl.when(j + 1 < NCHUNK)
      def _():
        @pl.when((j + 1) % 2 == 0)
        def _():
          pltpu.async_copy(table_hbm.at[idx_v.at[j + 1]], bufs[0], sems[0])

        @pl.when((j + 1) % 2 == 1)
        def _():
          pltpu.async_copy(table_hbm.at[idx_v.at[j + 1]], bufs[1], sems[1])

      @pl.when(j % 2 == 0)
      def _():
        pltpu.make_async_copy(table_hbm.at[idx_v.at[0]], bufs[0], sems[0]).wait()
        pltpu.sync_copy(bufs[0], out_hbm.at[pl.ds(base + j * CHUNK, CHUNK)])

      @pl.when(j % 2 == 1)
      def _():
        pltpu.make_async_copy(table_hbm.at[idx_v.at[0]], bufs[1], sems[1]).wait()
        pltpu.sync_copy(bufs[1], out_hbm.at[pl.ds(base + j * CHUNK, CHUNK)])

      return 0

    lax.fori_loop(0, NCHUNK, step, 0)

  return k


_gather_kernel = _make_kernel()


@jax.jit
def kernel(x, embedding):
  idx = x.reshape(ROWS // CHUNK, CHUNK).astype(jnp.int32)
  out = _gather_kernel(idx, embedding)
  return out.reshape(x.shape[0], x.shape[1], D)


# speedup vs baseline: 1.0207x; 1.0207x over previous
import functools

import jax
import jax.numpy as jnp
from jax import lax
from jax.experimental import pallas as pl
from jax.experimental.pallas import tpu as pltpu
from jax.experimental.pallas import tpu_sc as plsc

NC = 2
NS = 16
NW = NC * NS

ROWS = 4096 * 200
D = 64
CHUNK = 128
K = 5
GROUP_ROWS = K * CHUNK
ROWS_PER_W = ROWS // NW
NCHUNK = ROWS_PER_W // CHUNK
NGROUP = NCHUNK // K
NPAIR = NGROUP // 2


def _make_kernel():
  mesh = plsc.VectorSubcoreMesh(core_axis_name="c", subcore_axis_name="s")

  @functools.partial(
      pl.kernel,
      out_type=jax.ShapeDtypeStruct((ROWS, D), jnp.float32),
      mesh=mesh,
      compiler_params=pltpu.CompilerParams(use_tc_tiling_on_sc=False),
      scratch_types=[
          pltpu.VMEM((NCHUNK, CHUNK), jnp.int32),
          pltpu.VMEM((GROUP_ROWS, D), jnp.float32),
          pltpu.VMEM((GROUP_ROWS, D), jnp.float32),
          pltpu.SemaphoreType.DMA,
          pltpu.SemaphoreType.DMA,
          pltpu.SemaphoreType.DMA,
          pltpu.SemaphoreType.DMA,
      ],
  )
  def k(idx_hbm, table_hbm, out_hbm, idx_v, buf0, buf1, g0s, g1s, w0s, w1s):
    wid = lax.axis_index("s") * NC + lax.axis_index("c")
    base = wid * ROWS_PER_W
    pltpu.sync_copy(idx_hbm.at[pl.ds(wid * NCHUNK, NCHUNK)], idx_v)

    def run_group(i, g, buf, gsem, wsem):
      @pl.when(i > 0)
      def _():
        pltpu.make_async_copy(
            buf, out_hbm.at[pl.ds(base, GROUP_ROWS)], wsem).wait()

      for b in range(K):
        pltpu.async_copy(
            table_hbm.at[idx_v.at[g * K + b]],
            buf.at[pl.ds(b * CHUNK, CHUNK)], gsem)
      for b in range(K):
        pltpu.make_async_copy(
            table_hbm.at[idx_v.at[g * K + b]],
            buf.at[pl.ds(b * CHUNK, CHUNK)], gsem).wait()

      pltpu.async_copy(
          buf, out_hbm.at[pl.ds(base + g * GROUP_ROWS, GROUP_ROWS)], wsem)

    def step(i, _):
      run_group(i, 2 * i, buf0, g0s, w0s)
      run_group(i, 2 * i + 1, buf1, g1s, w1s)
      return 0

    lax.fori_loop(0, NPAIR, step, 0)

    pltpu.make_async_copy(buf0, out_hbm.at[pl.ds(base, GROUP_ROWS)], w0s).wait()
    pltpu.make_async_copy(buf1, out_hbm.at[pl.ds(base, GROUP_ROWS)], w1s).wait()

  return k


_gather_kernel = _make_kernel()


@jax.jit
def kernel(x, embedding):
  idx = x.reshape(ROWS // CHUNK, CHUNK).astype(jnp.int32)
  out = _gather_kernel(idx, embedding)
  return out.reshape(x.shape[0], x.shape[1], D)
